# baseline (speedup 1.0000x reference)
import functools

import jax
import jax.numpy as jnp
from jax import lax
from jax.experimental import pallas as pl
from jax.experimental.pallas import tpu as pltpu
from jax.experimental.pallas import tpu_sc as plsc

_N = 320000
_C = 128
_S = 64
_EPS = 1e-05

_NSC = 8192
_M = _N - _NSC
_NW = 32
_R = _NSC // _NW
_CH = 256
_NCHUNK = _R // _CH
_NGRP = _CH // 16

_B1 = 19488
_NB1 = _M // _B1
_B2 = 20000
_NB2 = _N // _B2


def _sc_partials_body(x_hbm, seg_hbm, sums_hbm, sumsq_hbm, cnt_hbm,
                      xv, idsv, accs, accq, accc):
    nc = 2
    wid = lax.axis_index("s") * nc + lax.axis_index("c")
    base = _M + wid * _R

    zero16 = jnp.zeros((16,), jnp.float32)

    def _zero_seg(s, _):
        for j in range(_C // 16):
            accs[s, pl.ds(16 * j, 16)] = zero16
            accq[s, pl.ds(16 * j, 16)] = zero16
        accc[s, pl.ds(0, 16)] = zero16
        return 0

    lax.fori_loop(0, _S, _zero_seg, 0)

    one_16th = jnp.full((16,), 1.0 / 16.0, jnp.float32)
    ones = jnp.ones((16,), jnp.float32)
    csl = pl.ds(0, 16)

    def _do_chunk(c, _):
        row0 = base + c * _CH
        pltpu.sync_copy(x_hbm.at[pl.ds(row0, _CH)], xv)
        pltpu.sync_copy(seg_hbm.at[pl.ds(row0, _CH)], idsv)

        def _do_group(g, _):
            r0 = g * 16
            segv = idsv[pl.ds(r0, 16)]
            s_first = segv[0]
            s_last = segv[15]

            @pl.when(s_first == s_last)
            def _fast():
                for j in range(_C // 16):
                    sl = pl.ds(16 * j, 16)
                    v = xv[r0, sl]
                    acc = v
                    accsq = v * v
                    for r in range(1, 16):
                        v = xv[r0 + r, sl]
                        acc = acc + v
                        accsq = accsq + v * v
                    accs[s_first, sl] = accs[s_first, sl] + acc
                    accq[s_first, sl] = accq[s_first, sl] + accsq
                accc[s_first, csl] = accc[s_first, csl] + ones

            @pl.when(s_first != s_last)
            def _slow():
                for r in range(16):
                    s = segv[r]
                    for j in range(_C // 16):
                        sl = pl.ds(16 * j, 16)
                        v = xv[r0 + r, sl]
                        accs[s, sl] = accs[s, sl] + v
                        accq[s, sl] = accq[s, sl] + v * v
                    accc[s, csl] = accc[s, csl] + one_16th

            return 0

        lax.fori_loop(0, _NGRP, _do_group, 0)
        return 0

    lax.fori_loop(0, _NCHUNK, _do_chunk, 0)

    pltpu.sync_copy(accs, sums_hbm.at[wid])
    pltpu.sync_copy(accq, sumsq_hbm.at[wid])
    pltpu.sync_copy(accc, cnt_hbm.at[wid])


def _sc_partials(x, seg):
    mesh = plsc.VectorSubcoreMesh(core_axis_name="c", subcore_axis_name="s")
    f32 = jnp.float32
    kern = functools.partial(
        pl.kernel,
        mesh=mesh,
        out_type=[
            jax.ShapeDtypeStruct((_NW, _S, _C), f32),
            jax.ShapeDtypeStruct((_NW, _S, _C), f32),
            jax.ShapeDtypeStruct((_NW, _S, 16), f32),
        ],
        scratch_types=[
            pltpu.VMEM((_CH, _C), f32),
            pltpu.VMEM((_CH,), jnp.int32),
            pltpu.VMEM((_S, _C), f32),
            pltpu.VMEM((_S, _C), f32),
            pltpu.VMEM((_S, 16), f32),
        ],
    )(_sc_partials_body)
    return kern(x, seg)


def _tc_acc_body(x_ref, seg_ref, acc_ref, out_ref):
    i = pl.program_id(0)

    @pl.when(i == 0)
    def _init():
        acc_ref[...] = jnp.zeros_like(acc_ref)

    x = x_ref[...]
    segs = seg_ref[0, 0, :]
    onehot_t = (lax.broadcasted_iota(jnp.int32, (_S, _B1), 0)
                == segs[None, :]).astype(jnp.bfloat16)
    y = jnp.concatenate([x, x * x, jnp.ones_like(x)],
                        axis=1).astype(jnp.bfloat16)
    acc_ref[...] += jnp.dot(onehot_t, y, preferred_element_type=jnp.float32)

    @pl.when(i == _NB1 - 1)
    def _emit():
        out_ref[...] = acc_ref[...]


def _combine_body(tc_ref, ss_ref, sq_ref, sc_ref, w_ref, b_ref, out_ref):
    tc = tc_ref[...]
    sums = tc[:, :_C] + jnp.sum(ss_ref[...], axis=0)
    sumsq = tc[:, _C:2 * _C] + jnp.sum(sq_ref[...], axis=0)
    cnt = tc[:, 2 * _C:] + jnp.sum(sc_ref[...], axis=(0, 2))[:, None]
    cnt = jnp.maximum(cnt, 1.0)
    mean = sums / cnt
    var = jnp.maximum(sumsq / cnt - mean * mean, 0.0)
    instd = lax.rsqrt(var + _EPS)
    scale = instd * w_ref[0, :][None, :]
    shift = b_ref[0, :][None, :] - mean * scale
    out_ref[...] = jnp.concatenate([scale, shift], axis=1)


def _apply_body(x_ref, seg_ref, st_ref, o_ref):
    segs = seg_ref[0, 0, :]
    onehot = (segs[:, None]
              == lax.broadcasted_iota(jnp.int32, (_B2, _S), 1)).astype(jnp.bfloat16)
    rows = jnp.dot(onehot, st_ref[...].astype(jnp.bfloat16),
                   preferred_element_type=jnp.float32)
    o_ref[...] = x_ref[...] * rows[:, :_C] + rows[:, _C:]


@jax.jit
def kernel(x, segment_ids, weight, bias):
    seg = segment_ids.astype(jnp.int32)
    seg3_head = seg[:_M].reshape(_NB1, 1, _B1)
    seg3_all = seg.reshape(_NB2, 1, _B2)

    sc_sums, sc_sumsq, sc_cnt = _sc_partials(x, seg)

    tc_acc = pl.pallas_call(
        _tc_acc_body,
        grid=(_NB1,),
        in_specs=[
            pl.BlockSpec((_B1, _C), lambda i: (i, 0)),
            pl.BlockSpec((1, 1, _B1), lambda i: (i, 0, 0)),
        ],
        out_specs=pl.BlockSpec((_S, 3 * _C), lambda i: (0, 0)),
        out_shape=jax.ShapeDtypeStruct((_S, 3 * _C), jnp.float32),
        scratch_shapes=[pltpu.VMEM((_S, 3 * _C), jnp.float32)],
    )(x, seg3_head)

    stats = pl.pallas_call(
        _combine_body,
        grid=(1,),
        in_specs=[
            pl.BlockSpec((_S, 3 * _C), lambda i: (0, 0)),
            pl.BlockSpec((_NW, _S, _C), lambda i: (0, 0, 0)),
            pl.BlockSpec((_NW, _S, _C), lambda i: (0, 0, 0)),
            pl.BlockSpec((_NW, _S, 16), lambda i: (0, 0, 0)),
            pl.BlockSpec((1, _C), lambda i: (0, 0)),
            pl.BlockSpec((1, _C), lambda i: (0, 0)),
        ],
        out_specs=pl.BlockSpec((_S, 2 * _C), lambda i: (0, 0)),
        out_shape=jax.ShapeDtypeStruct((_S, 2 * _C), jnp.float32),
    )(tc_acc, sc_sums, sc_sumsq, sc_cnt, weight, bias)

    out = pl.pallas_call(
        _apply_body,
        grid=(_NB2,),
        in_specs=[
            pl.BlockSpec((_B2, _C), lambda i: (i, 0)),
            pl.BlockSpec((1, 1, _B2), lambda i: (i, 0, 0)),
            pl.BlockSpec((_S, 2 * _C), lambda i: (0, 0)),
        ],
        out_specs=pl.BlockSpec((_B2, _C), lambda i: (i, 0)),
        out_shape=jax.ShapeDtypeStruct((_N, _C), jnp.float32),
    )(x, seg3_all, stats)
    return out


# speedup vs baseline: 1.0085x; 1.0085x over previous
import functools

import jax
import jax.numpy as jnp
from jax import lax
from jax.experimental import pallas as pl
from jax.experimental.pallas import tpu as pltpu
from jax.experimental.pallas import tpu_sc as plsc

_N = 320000
_C = 128
_S = 64
_EPS = 1e-05

_NSC = 12800
_M = _N - _NSC
_NW = 32
_R = _NSC // _NW
_CH = 400
_NCHUNK = _R // _CH
_NGRP = _CH // 16

_B1 = 19200
_NB1 = _M // _B1
_B2 = 20000
_NB2 = _N // _B2


def _sc_partials_body(x_hbm, seg_hbm, sums_hbm, sumsq_hbm, cnt_hbm,
                      xv, idsv, accs, accq, accc):
    nc = 2
    wid = lax.axis_index("s") * nc + lax.axis_index("c")
    base = _M + wid * _R

    zero16 = jnp.zeros((16,), jnp.float32)

    def _zero_seg(s, _):
        for j in range(_C // 16):
            accs[s, pl.ds(16 * j, 16)] = zero16
            accq[s, pl.ds(16 * j, 16)] = zero16
        accc[s, pl.ds(0, 16)] = zero16
        return 0

    lax.fori_loop(0, _S, _zero_seg, 0)

    one_16th = jnp.full((16,), 1.0 / 16.0, jnp.float32)
    ones = jnp.ones((16,), jnp.float32)
    csl = pl.ds(0, 16)

    def _do_chunk(c, _):
        row0 = base + c * _CH
        pltpu.sync_copy(x_hbm.at[pl.ds(row0, _CH)], xv)
        pltpu.sync_copy(seg_hbm.at[pl.ds(row0, _CH)], idsv)

        def _do_group(g, _):
            r0 = g * 16
            segv = idsv[pl.ds(r0, 16)]
            s_first = segv[0]
            s_last = segv[15]

            @pl.when(s_first == s_last)
            def _fast():
                for j in range(_C // 16):
                    sl = pl.ds(16 * j, 16)
                    v = xv[r0, sl]
                    acc = v
                    accsq = v * v
                    for r in range(1, 16):
                        v = xv[r0 + r, sl]
                        acc = acc + v
                        accsq = accsq + v * v
                    accs[s_first, sl] = accs[s_first, sl] + acc
                    accq[s_first, sl] = accq[s_first, sl] + accsq
                accc[s_first, csl] = accc[s_first, csl] + ones

            @pl.when(s_first != s_last)
            def _slow():
                for r in range(16):
                    s = segv[r]
                    for j in range(_C // 16):
                        sl = pl.ds(16 * j, 16)
                        v = xv[r0 + r, sl]
                        accs[s, sl] = accs[s, sl] + v
                        accq[s, sl] = accq[s, sl] + v * v
                    accc[s, csl] = accc[s, csl] + one_16th

            return 0

        lax.fori_loop(0, _NGRP, _do_group, 0)
        return 0

    lax.fori_loop(0, _NCHUNK, _do_chunk, 0)

    pltpu.sync_copy(accs, sums_hbm.at[wid])
    pltpu.sync_copy(accq, sumsq_hbm.at[wid])
    pltpu.sync_copy(accc, cnt_hbm.at[wid])


def _sc_partials(x, seg):
    mesh = plsc.VectorSubcoreMesh(core_axis_name="c", subcore_axis_name="s")
    f32 = jnp.float32
    kern = functools.partial(
        pl.kernel,
        mesh=mesh,
        out_type=[
            jax.ShapeDtypeStruct((_NW, _S, _C), f32),
            jax.ShapeDtypeStruct((_NW, _S, _C), f32),
            jax.ShapeDtypeStruct((_NW, _S, 16), f32),
        ],
        scratch_types=[
            pltpu.VMEM((_CH, _C), f32),
            pltpu.VMEM((_CH,), jnp.int32),
            pltpu.VMEM((_S, _C), f32),
            pltpu.VMEM((_S, _C), f32),
            pltpu.VMEM((_S, 16), f32),
        ],
    )(_sc_partials_body)
    return kern(x, seg)


def _tc_acc_body(x_ref, seg_ref, acc_ref, out_ref):
    i = pl.program_id(0)

    @pl.when(i == 0)
    def _init():
        acc_ref[...] = jnp.zeros_like(acc_ref)

    x = x_ref[...]
    segs = seg_ref[0, 0, :]
    onehot_t = (lax.broadcasted_iota(jnp.int32, (_S, _B1), 0)
                == segs[None, :]).astype(jnp.bfloat16)
    y = jnp.concatenate([x, x * x, jnp.ones_like(x)],
                        axis=1).astype(jnp.bfloat16)
    acc_ref[...] += jnp.dot(onehot_t, y, preferred_element_type=jnp.float32)

    @pl.when(i == _NB1 - 1)
    def _emit():
        out_ref[...] = acc_ref[...]


def _combine_body(tc_ref, ss_ref, sq_ref, sc_ref, w_ref, b_ref, out_ref):
    tc = tc_ref[...]
    sums = tc[:, :_C] + jnp.sum(ss_ref[...], axis=0)
    sumsq = tc[:, _C:2 * _C] + jnp.sum(sq_ref[...], axis=0)
    cnt = tc[:, 2 * _C:] + jnp.sum(sc_ref[...], axis=(0, 2))[:, None]
    cnt = jnp.maximum(cnt, 1.0)
    mean = sums / cnt
    var = jnp.maximum(sumsq / cnt - mean * mean, 0.0)
    instd = lax.rsqrt(var + _EPS)
    scale = instd * w_ref[0, :][None, :]
    shift = b_ref[0, :][None, :] - mean * scale
    out_ref[...] = jnp.concatenate([scale, shift], axis=1)


def _apply_body(x_ref, seg_ref, st_ref, o_ref):
    segs = seg_ref[0, 0, :]
    onehot = (segs[:, None]
              == lax.broadcasted_iota(jnp.int32, (_B2, _S), 1)).astype(jnp.bfloat16)
    rows = jnp.dot(onehot, st_ref[...].astype(jnp.bfloat16),
                   preferred_element_type=jnp.float32)
    o_ref[...] = x_ref[...] * rows[:, :_C] + rows[:, _C:]


@jax.jit
def kernel(x, segment_ids, weight, bias):
    seg = segment_ids.astype(jnp.int32)
    seg3_head = seg[:_M].reshape(_NB1, 1, _B1)
    seg3_all = seg.reshape(_NB2, 1, _B2)

    sc_sums, sc_sumsq, sc_cnt = _sc_partials(x, seg)

    tc_acc = pl.pallas_call(
        _tc_acc_body,
        grid=(_NB1,),
        in_specs=[
            pl.BlockSpec((_B1, _C), lambda i: (i, 0)),
            pl.BlockSpec((1, 1, _B1), lambda i: (i, 0, 0)),
        ],
        out_specs=pl.BlockSpec((_S, 3 * _C), lambda i: (0, 0)),
        out_shape=jax.ShapeDtypeStruct((_S, 3 * _C), jnp.float32),
        scratch_shapes=[pltpu.VMEM((_S, 3 * _C), jnp.float32)],
    )(x, seg3_head)

    stats = pl.pallas_call(
        _combine_body,
        grid=(1,),
        in_specs=[
            pl.BlockSpec((_S, 3 * _C), lambda i: (0, 0)),
            pl.BlockSpec((_NW, _S, _C), lambda i: (0, 0, 0)),
            pl.BlockSpec((_NW, _S, _C), lambda i: (0, 0, 0)),
            pl.BlockSpec((_NW, _S, 16), lambda i: (0, 0, 0)),
            pl.BlockSpec((1, _C), lambda i: (0, 0)),
            pl.BlockSpec((1, _C), lambda i: (0, 0)),
        ],
        out_specs=pl.BlockSpec((_S, 2 * _C), lambda i: (0, 0)),
        out_shape=jax.ShapeDtypeStruct((_S, 2 * _C), jnp.float32),
    )(tc_acc, sc_sums, sc_sumsq, sc_cnt, weight, bias)

    out = pl.pallas_call(
        _apply_body,
        grid=(_NB2,),
        in_specs=[
            pl.BlockSpec((_B2, _C), lambda i: (i, 0)),
            pl.BlockSpec((1, 1, _B2), lambda i: (i, 0, 0)),
            pl.BlockSpec((_S, 2 * _C), lambda i: (0, 0)),
        ],
        out_specs=pl.BlockSpec((_B2, _C), lambda i: (i, 0)),
        out_shape=jax.ShapeDtypeStruct((_N, _C), jnp.float32),
    )(x, seg3_all, stats)
    return out


# speedup vs baseline: 1.0103x; 1.0018x over previous
import functools

import jax
import jax.numpy as jnp
from jax import lax
from jax.experimental import pallas as pl
from jax.experimental.pallas import tpu as pltpu
from jax.experimental.pallas import tpu_sc as plsc

_N = 320000
_C = 128
_S = 64
_EPS = 1e-05

_NSC = 12800
_M = _N - _NSC
_NW = 32
_R = _NSC // _NW
_CH = 400
_NCHUNK = _R // _CH
_NGRP = _CH // 16

_B1 = 19200
_NB1 = _M // _B1
_B2 = 20000
_NB2 = _N // _B2


def _sc_partials_body(x_hbm, seg_hbm, sums_hbm, sumsq_hbm, cnt_hbm,
                      xv, idsv, accs, accq, accc):
    nc = 2
    wid = lax.axis_index("s") * nc + lax.axis_index("c")
    base = _M + wid * _R

    zero16 = jnp.zeros((16,), jnp.float32)

    def _zero_seg(s, _):
        for j in range(_C // 16):
            accs[s, pl.ds(16 * j, 16)] = zero16
            accq[s, pl.ds(16 * j, 16)] = zero16
        accc[s, pl.ds(0, 16)] = zero16
        return 0

    lax.fori_loop(0, _S, _zero_seg, 0)

    one_16th = jnp.full((16,), 1.0 / 16.0, jnp.float32)
    ones = jnp.ones((16,), jnp.float32)
    csl = pl.ds(0, 16)

    def _do_chunk(c, _):
        row0 = base + c * _CH
        pltpu.sync_copy(x_hbm.at[pl.ds(row0, _CH)], xv)
        pltpu.sync_copy(seg_hbm.at[pl.ds(row0, _CH)], idsv)

        def _do_group(g, _):
            r0 = g * 16
            segv = idsv[pl.ds(r0, 16)]
            s_first = segv[0]
            s_last = segv[15]

            @pl.when(s_first == s_last)
            def _fast():
                for j in range(_C // 16):
                    sl = pl.ds(16 * j, 16)
                    v = xv[r0, sl]
                    acc = v
                    accsq = v * v
                    for r in range(1, 16):
                        v = xv[r0 + r, sl]
                        acc = acc + v
                        accsq = accsq + v * v
                    accs[s_first, sl] = accs[s_first, sl] + acc
                    accq[s_first, sl] = accq[s_first, sl] + accsq
                accc[s_first, csl] = accc[s_first, csl] + ones

            @pl.when(s_first != s_last)
            def _slow():
                for r in range(16):
                    s = segv[r]
                    for j in range(_C // 16):
                        sl = pl.ds(16 * j, 16)
                        v = xv[r0 + r, sl]
                        accs[s, sl] = accs[s, sl] + v
                        accq[s, sl] = accq[s, sl] + v * v
                    accc[s, csl] = accc[s, csl] + one_16th

            return 0

        lax.fori_loop(0, _NGRP, _do_group, 0)
        return 0

    lax.fori_loop(0, _NCHUNK, _do_chunk, 0)

    pltpu.sync_copy(accs, sums_hbm.at[wid])
    pltpu.sync_copy(accq, sumsq_hbm.at[wid])
    pltpu.sync_copy(accc, cnt_hbm.at[wid])


def _sc_partials(x, seg):
    mesh = plsc.VectorSubcoreMesh(core_axis_name="c", subcore_axis_name="s")
    f32 = jnp.float32
    kern = functools.partial(
        pl.kernel,
        mesh=mesh,
        out_type=[
            jax.ShapeDtypeStruct((_NW, _S, _C), f32),
            jax.ShapeDtypeStruct((_NW, _S, _C), f32),
            jax.ShapeDtypeStruct((_NW, _S, 16), f32),
        ],
        scratch_types=[
            pltpu.VMEM((_CH, _C), f32),
            pltpu.VMEM((_CH,), jnp.int32),
            pltpu.VMEM((_S, _C), f32),
            pltpu.VMEM((_S, _C), f32),
            pltpu.VMEM((_S, 16), f32),
        ],
        cost_estimate=pl.CostEstimate(
            flops=4 * _NSC * _C,
            bytes_accessed=_NSC * (_C * 4 + 4) + _NW * _S * (2 * _C + 16) * 4,
            transcendentals=0,
        ),
    )(_sc_partials_body)
    return kern(x, seg)


def _tc_acc_body(x_ref, seg_ref, acc_ref, out_ref):
    i = pl.program_id(0)

    @pl.when(i == 0)
    def _init():
        acc_ref[...] = jnp.zeros_like(acc_ref)

    x = x_ref[...]
    segs = seg_ref[0, 0, :]
    onehot_t = (lax.broadcasted_iota(jnp.int32, (_S, _B1), 0)
                == segs[None, :]).astype(jnp.bfloat16)
    y = jnp.concatenate([x, x * x, jnp.ones_like(x)],
                        axis=1).astype(jnp.bfloat16)
    acc_ref[...] += jnp.dot(onehot_t, y, preferred_element_type=jnp.float32)

    @pl.when(i == _NB1 - 1)
    def _emit():
        out_ref[...] = acc_ref[...]


def _combine_body(tc_ref, ss_ref, sq_ref, sc_ref, w_ref, b_ref, out_ref):
    tc = tc_ref[...]
    sums = tc[:, :_C] + jnp.sum(ss_ref[...], axis=0)
    sumsq = tc[:, _C:2 * _C] + jnp.sum(sq_ref[...], axis=0)
    cnt = tc[:, 2 * _C:] + jnp.sum(sc_ref[...], axis=(0, 2))[:, None]
    cnt = jnp.maximum(cnt, 1.0)
    mean = sums / cnt
    var = jnp.maximum(sumsq / cnt - mean * mean, 0.0)
    instd = lax.rsqrt(var + _EPS)
    scale = instd * w_ref[0, :][None, :]
    shift = b_ref[0, :][None, :] - mean * scale
    out_ref[...] = jnp.concatenate([scale, shift], axis=1)


def _apply_body(x_ref, seg_ref, st_ref, o_ref):
    segs = seg_ref[0, 0, :]
    onehot = (segs[:, None]
              == lax.broadcasted_iota(jnp.int32, (_B2, _S), 1)).astype(jnp.bfloat16)
    rows = jnp.dot(onehot, st_ref[...].astype(jnp.bfloat16),
                   preferred_element_type=jnp.float32)
    o_ref[...] = x_ref[...] * rows[:, :_C] + rows[:, _C:]


@jax.jit
def kernel(x, segment_ids, weight, bias):
    seg = segment_ids.astype(jnp.int32)
    seg3_head = seg[:_M].reshape(_NB1, 1, _B1)
    seg3_all = seg.reshape(_NB2, 1, _B2)

    sc_sums, sc_sumsq, sc_cnt = _sc_partials(x, seg)

    tc_acc = pl.pallas_call(
        _tc_acc_body,
        grid=(_NB1,),
        in_specs=[
            pl.BlockSpec((_B1, _C), lambda i: (i, 0)),
            pl.BlockSpec((1, 1, _B1), lambda i: (i, 0, 0)),
        ],
        out_specs=pl.BlockSpec((_S, 3 * _C), lambda i: (0, 0)),
        out_shape=jax.ShapeDtypeStruct((_S, 3 * _C), jnp.float32),
        scratch_shapes=[pltpu.VMEM((_S, 3 * _C), jnp.float32)],
    )(x, seg3_head)

    stats = pl.pallas_call(
        _combine_body,
        grid=(1,),
        in_specs=[
            pl.BlockSpec((_S, 3 * _C), lambda i: (0, 0)),
            pl.BlockSpec((_NW, _S, _C), lambda i: (0, 0, 0)),
            pl.BlockSpec((_NW, _S, _C), lambda i: (0, 0, 0)),
            pl.BlockSpec((_NW, _S, 16), lambda i: (0, 0, 0)),
            pl.BlockSpec((1, _C), lambda i: (0, 0)),
            pl.BlockSpec((1, _C), lambda i: (0, 0)),
        ],
        out_specs=pl.BlockSpec((_S, 2 * _C), lambda i: (0, 0)),
        out_shape=jax.ShapeDtypeStruct((_S, 2 * _C), jnp.float32),
    )(tc_acc, sc_sums, sc_sumsq, sc_cnt, weight, bias)

    out = pl.pallas_call(
        _apply_body,
        grid=(_NB2,),
        in_specs=[
            pl.BlockSpec((_B2, _C), lambda i: (i, 0)),
            pl.BlockSpec((1, 1, _B2), lambda i: (i, 0, 0)),
            pl.BlockSpec((_S, 2 * _C), lambda i: (0, 0)),
        ],
        out_specs=pl.BlockSpec((_B2, _C), lambda i: (i, 0)),
        out_shape=jax.ShapeDtypeStruct((_N, _C), jnp.float32),
    )(x, seg3_all, stats)
    return out
